# baseline (speedup 1.0000x reference)
import functools

import jax
import jax.numpy as jnp
from jax import lax
from jax.experimental import pallas as pl
from jax.experimental.pallas import tpu as pltpu
from jax.experimental.pallas import tpu_sc as plsc

B = 4096
D_DENSE = 13
ED = 128
V = 100000
F = 26

NC = 2
NS = 16
NW = NC * NS
SLOTS = 32
BPC = 4
CH = BPC * F
NSPLIT = 2


def _scatter_chunk(buf, out_hbm, row0, sem):
    for j in range(BPC):
        pltpu.async_copy(buf.at[pl.ds(j * F, F)],
                         out_hbm.at[pl.ds(row0 + j * SLOTS, F)], sem)
    for j in range(BPC):
        pltpu.make_async_copy(buf.at[pl.ds(j * F, F)],
                              out_hbm.at[pl.ds(row0 + j * SLOTS, F)],
                              sem).wait()


def _sc_gather_body(nch, table_hbm, idx_hbm, out_hbm, idx_v, buf0, buf1,
                    sem0, sem1, osem):
    wid = lax.axis_index("s") * NC + lax.axis_index("c")
    base = wid * (nch * BPC * SLOTS)
    pltpu.sync_copy(idx_hbm.at[wid], idx_v)
    pltpu.async_copy(table_hbm.at[idx_v.at[0]], buf0, sem0)
    cstride = BPC * SLOTS

    def body(i, carry):
        c0 = 2 * i
        pltpu.async_copy(table_hbm.at[idx_v.at[c0 + 1]], buf1, sem1)
        pltpu.make_async_copy(table_hbm.at[idx_v.at[0]], buf0, sem0).wait()
        _scatter_chunk(buf0, out_hbm, base + c0 * cstride, osem)

        @pl.when(c0 + 2 < nch)
        def _():
            pltpu.async_copy(table_hbm.at[idx_v.at[c0 + 2]], buf0, sem0)

        pltpu.make_async_copy(table_hbm.at[idx_v.at[0]], buf1, sem1).wait()
        _scatter_chunk(buf1, out_hbm, base + (c0 + 1) * cstride, osem)
        return carry

    lax.fori_loop(0, nch // 2, body, 0)


@functools.lru_cache(maxsize=None)
def _make_sc_gather(nbatch):
    nch = nbatch // NW // BPC
    mesh = plsc.VectorSubcoreMesh(
        core_axis_name="c", subcore_axis_name="s",
        num_cores=NC, num_subcores=NS)
    return pl.kernel(
        functools.partial(_sc_gather_body, nch),
        out_type=jax.ShapeDtypeStruct((nbatch * SLOTS, ED), jnp.float32),
        mesh=mesh,
        compiler_params=pltpu.CompilerParams(use_tc_tiling_on_sc=False),
        scratch_types=[
            pltpu.VMEM((nch, CH), jnp.int32),
            pltpu.VMEM((CH, ED), jnp.float32),
            pltpu.VMEM((CH, ED), jnp.float32),
            pltpu.SemaphoreType.DMA,
            pltpu.SemaphoreType.DMA,
            pltpu.SemaphoreType.DMA,
        ],
    )


def _tc_body(xd, embs, bw0, bb0, bw1, bb1, bw2, bb2, tw0d, w2, tb0, tw1,
             tb1, tw2, tb2, out):
    x = xd[...]
    d = jnp.maximum(jnp.dot(x, bw0[...]) + bb0[...], 0.0)
    d = jnp.maximum(jnp.dot(d, bw1[...]) + bb1[...], 0.0)
    d = jnp.maximum(jnp.dot(d, bw2[...]) + bb2[...], 0.0)
    e27 = embs[...][:, :F + 1, :]
    slot = lax.broadcasted_iota(jnp.int32, (1, F + 1, 1), 1)
    bb = d.shape[0]
    all3 = jnp.where(slot < F, e27,
                     jnp.broadcast_to(d[:, None, :], (bb, F + 1, ED)))
    acc = jnp.dot(d, tw0d[...]) + tb0[...]
    all3b = all3.astype(jnp.bfloat16)
    s = lax.dot_general(all3b, all3b,
                        dimension_numbers=(((2,), (2,)), ((0,), (0,))),
                        preferred_element_type=jnp.float32)
    for n in range(F + 1):
        acc = acc + jnp.dot(s[:, n, :], w2[n])
    t = jnp.maximum(acc, 0.0)
    t = jnp.maximum(jnp.dot(t, tw1[...]) + tb1[...], 0.0)
    o = jnp.dot(t, tw2[...]) + tb2[...]
    out[...] = 1.0 / (1.0 + jnp.exp(-o))


def kernel(dense_x, sparse_x, sparse_offsets, tables, bw0, bb0, bw1, bb1,
           bw2, bb2, tw0, tb0, tw1, tb1, tw2, tb2):
    del sparse_offsets
    offs = jnp.arange(F, dtype=jnp.int32) * V
    idx_all = sparse_x + offs[None, :]
    table2d = tables.reshape(F * V, ED)
    bs = B // NSPLIT
    nch = bs // NW // BPC
    sc_gather = _make_sc_gather(bs)
    embs_parts = []
    for k in range(NSPLIT):
        idx_k = lax.slice_in_dim(idx_all, k * bs, (k + 1) * bs, axis=0)
        gathered = sc_gather(table2d, idx_k.reshape(NW, nch, CH))
        embs_parts.append(gathered.reshape(bs, SLOTS, ED))

    tri = tw0[ED:]
    rows = []
    for s in range(F):
        n = s + 1
        off = 26 * n - (n * (n - 1)) // 2
        blk = lax.slice_in_dim(tri, off, off + F - n, axis=0)
        rows.append(jnp.pad(blk, ((s + 1, 1), (0, 0))))
    rows.append(jnp.pad(lax.slice_in_dim(tri, 0, F, axis=0),
                        ((0, 1), (0, 0))))
    w2 = jnp.stack(rows, axis=0)
    tw0d = tw0[:ED]

    Bb = 1024
    nb = bs // Bb
    full = lambda *s: pl.BlockSpec(s, lambda i: tuple(0 for _ in s))
    dense_call = pl.pallas_call(
        _tc_body,
        grid=(nb,),
        in_specs=[
            pl.BlockSpec((Bb, D_DENSE), lambda i: (i, 0)),
            pl.BlockSpec((Bb, SLOTS, ED), lambda i: (i, 0, 0)),
            full(D_DENSE, 512),
            full(1, 512),
            full(512, 256),
            full(1, 256),
            full(256, ED),
            full(1, ED),
            full(ED, 512),
            full(F + 1, F + 1, 512),
            full(1, 512),
            full(512, 256),
            full(1, 256),
            full(256, 1),
            full(1, 1),
        ],
        out_specs=pl.BlockSpec((Bb, 1), lambda i: (i, 0)),
        out_shape=jax.ShapeDtypeStruct((bs, 1), jnp.float32),
        compiler_params=pltpu.CompilerParams(
            vmem_limit_bytes=128 * 1024 * 1024),
    )
    outs = []
    for k in range(NSPLIT):
        dense_k = lax.slice_in_dim(dense_x, k * bs, (k + 1) * bs, axis=0)
        outs.append(dense_call(
            dense_k, embs_parts[k], bw0, bb0.reshape(1, 512), bw1,
            bb1.reshape(1, 256), bw2, bb2.reshape(1, ED), tw0d, w2,
            tb0.reshape(1, 512), tw1, tb1.reshape(1, 256), tw2,
            tb2.reshape(1, 1)))
    return jnp.concatenate(outs, axis=0)


# speedup vs baseline: 1.2013x; 1.2013x over previous
import functools

import jax
import jax.numpy as jnp
from jax import lax
from jax.experimental import pallas as pl
from jax.experimental.pallas import tpu as pltpu
from jax.experimental.pallas import tpu_sc as plsc

B = 4096
D_DENSE = 13
ED = 128
V = 100000
F = 26

NC = 2
NS = 16
NW = NC * NS
SLOTS = 32
BPC = 4
CH = BPC * F
NSPLIT = 2


def _scatter_chunk(buf, out_hbm, row0, sem):
    for j in range(BPC):
        pltpu.async_copy(buf.at[pl.ds(j * F, F)],
                         out_hbm.at[pl.ds(row0 + j * SLOTS, F)], sem)
    for j in range(BPC):
        pltpu.make_async_copy(buf.at[pl.ds(j * F, F)],
                              out_hbm.at[pl.ds(row0 + j * SLOTS, F)],
                              sem).wait()


def _sc_gather_body(nch, table_hbm, idx_hbm, out_hbm, idx_v, buf0, buf1,
                    sem0, sem1, osem):
    wid = lax.axis_index("s") * NC + lax.axis_index("c")
    base = wid * (nch * BPC * SLOTS)
    pltpu.sync_copy(idx_hbm.at[wid], idx_v)
    pltpu.async_copy(table_hbm.at[idx_v.at[0]], buf0, sem0)
    cstride = BPC * SLOTS

    def body(i, carry):
        c0 = 2 * i
        pltpu.async_copy(table_hbm.at[idx_v.at[c0 + 1]], buf1, sem1)
        pltpu.make_async_copy(table_hbm.at[idx_v.at[0]], buf0, sem0).wait()
        _scatter_chunk(buf0, out_hbm, base + c0 * cstride, osem)

        @pl.when(c0 + 2 < nch)
        def _():
            pltpu.async_copy(table_hbm.at[idx_v.at[c0 + 2]], buf0, sem0)

        pltpu.make_async_copy(table_hbm.at[idx_v.at[0]], buf1, sem1).wait()
        _scatter_chunk(buf1, out_hbm, base + (c0 + 1) * cstride, osem)
        return carry

    lax.fori_loop(0, nch // 2, body, 0)


@functools.lru_cache(maxsize=None)
def _make_sc_gather(nbatch):
    nch = nbatch // NW // BPC
    mesh = plsc.VectorSubcoreMesh(
        core_axis_name="c", subcore_axis_name="s",
        num_cores=NC, num_subcores=NS)
    return pl.kernel(
        functools.partial(_sc_gather_body, nch),
        out_type=jax.ShapeDtypeStruct((nbatch * SLOTS, ED), jnp.float32),
        mesh=mesh,
        compiler_params=pltpu.CompilerParams(use_tc_tiling_on_sc=False),
        scratch_types=[
            pltpu.VMEM((nch, CH), jnp.int32),
            pltpu.VMEM((CH, ED), jnp.float32),
            pltpu.VMEM((CH, ED), jnp.float32),
            pltpu.SemaphoreType.DMA,
            pltpu.SemaphoreType.DMA,
            pltpu.SemaphoreType.DMA,
        ],
    )


def _tc_body(xd, embs, bw0, bb0, bw1, bb1, bw2, bb2, tw0d, w2, tb0, tw1,
             tb1, tw2, tb2, out):
    x = xd[...]
    d = jnp.maximum(jnp.dot(x, bw0[...]) + bb0[...], 0.0)
    d = jnp.maximum(jnp.dot(d, bw1[...]) + bb1[...], 0.0)
    d = jnp.maximum(jnp.dot(d, bw2[...]) + bb2[...], 0.0)
    e27 = embs[...][:, :F + 1, :]
    slot = lax.broadcasted_iota(jnp.int32, (1, F + 1, 1), 1)
    bb = d.shape[0]
    all3 = jnp.where(slot < F, e27,
                     jnp.broadcast_to(d[:, None, :], (bb, F + 1, ED)))
    acc = jnp.dot(d, tw0d[...]) + tb0[...]
    all3b = all3.astype(jnp.bfloat16)
    s = lax.dot_general(all3b, all3b,
                        dimension_numbers=(((2,), (2,)), ((0,), (0,))),
                        preferred_element_type=jnp.float32)
    for n in range(F + 1):
        acc = acc + jnp.dot(s[:, n, :], w2[n])
    t = jnp.maximum(acc, 0.0)
    t = jnp.maximum(jnp.dot(t, tw1[...]) + tb1[...], 0.0)
    o = jnp.dot(t, tw2[...]) + tb2[...]
    out[...] = 1.0 / (1.0 + jnp.exp(-o))


def kernel(dense_x, sparse_x, sparse_offsets, tables, bw0, bb0, bw1, bb1,
           bw2, bb2, tw0, tb0, tw1, tb1, tw2, tb2):
    del sparse_offsets
    offs = jnp.arange(F, dtype=jnp.int32) * V
    idx_all = sparse_x + offs[None, :]
    table2d = tables.reshape(F * V, ED)
    bs = B // NSPLIT
    nch = bs // NW // BPC
    sc_gather = _make_sc_gather(bs)
    embs_parts = []
    for k in range(NSPLIT):
        idx_k = lax.slice_in_dim(idx_all, k * bs, (k + 1) * bs, axis=0)
        gathered = sc_gather(table2d, idx_k.reshape(NW, nch, CH))
        embs_parts.append(gathered.reshape(bs, SLOTS, ED))

    tri = tw0[ED:]
    rows = []
    for s in range(F):
        n = s + 1
        off = 26 * n - (n * (n - 1)) // 2
        blk = lax.slice_in_dim(tri, off, off + F - n, axis=0)
        rows.append(jnp.pad(blk, ((s + 1, 1), (0, 0))))
    rows.append(jnp.pad(lax.slice_in_dim(tri, 0, F, axis=0),
                        ((0, 1), (0, 0))))
    w2 = jnp.stack(rows, axis=0)
    tw0d = tw0[:ED]

    Bb = 512
    nb = bs // Bb
    full = lambda *s: pl.BlockSpec(s, lambda i: tuple(0 for _ in s))
    dense_call = pl.pallas_call(
        _tc_body,
        grid=(nb,),
        in_specs=[
            pl.BlockSpec((Bb, D_DENSE), lambda i: (i, 0)),
            pl.BlockSpec((Bb, SLOTS, ED), lambda i: (i, 0, 0)),
            full(D_DENSE, 512),
            full(1, 512),
            full(512, 256),
            full(1, 256),
            full(256, ED),
            full(1, ED),
            full(ED, 512),
            full(F + 1, F + 1, 512),
            full(1, 512),
            full(512, 256),
            full(1, 256),
            full(256, 1),
            full(1, 1),
        ],
        out_specs=pl.BlockSpec((Bb, 1), lambda i: (i, 0)),
        out_shape=jax.ShapeDtypeStruct((bs, 1), jnp.float32),
    )
    outs = []
    for k in range(NSPLIT):
        dense_k = lax.slice_in_dim(dense_x, k * bs, (k + 1) * bs, axis=0)
        outs.append(dense_call(
            dense_k, embs_parts[k], bw0, bb0.reshape(1, 512), bw1,
            bb1.reshape(1, 256), bw2, bb2.reshape(1, ED), tw0d, w2,
            tb0.reshape(1, 512), tw1, tb1.reshape(1, 256), tw2,
            tb2.reshape(1, 1)))
    return jnp.concatenate(outs, axis=0)
